# baseline (speedup 1.0000x reference)
import jax
import jax.numpy as jnp
import numpy as np
from jax.experimental import pallas as pl
from jax.experimental.pallas import tpu as pltpu

_EPS = 1e-5
_DIM = 4
_TRIL_R, _TRIL_C = np.tril_indices(_DIM)
_CORES = 2
_ROWS_BLK = 256


def _stats_kernel(x_ref, s_ref):
    j = pl.program_id(1)
    xb = x_ref[...]
    lanes = xb.shape[1]
    r1 = pltpu.roll(xb, lanes - 1, axis=1)
    r2 = pltpu.roll(xb, lanes - 2, axis=1)
    r3 = pltpu.roll(xb, lanes - 3, axis=1)
    rows = jnp.concatenate([
        jnp.sum(xb, axis=0, keepdims=True),
        jnp.sum(xb * xb, axis=0, keepdims=True),
        jnp.sum(xb * r1, axis=0, keepdims=True),
        jnp.sum(xb * r2, axis=0, keepdims=True),
        jnp.sum(xb * r3, axis=0, keepdims=True),
        jnp.zeros((3, lanes), jnp.float32),
    ], axis=0)

    @pl.when(j == 0)
    def _():
        s_ref[0] = rows

    @pl.when(j > 0)
    def _():
        s_ref[0] = s_ref[0] + rows


def _make_solve_kernel(batch):
    inv_b = 1.0 / batch

    def _solve_kernel(p_ref, q_ref):
        def row(k):
            return p_ref[k:k + 1, :]

        m = [row(c) * inv_b for c in range(4)]
        cov = {}
        for k, (r, c) in enumerate(zip(_TRIL_R, _TRIL_C)):
            e = row(4 + k) * inv_b - m[r] * m[c]
            if r == c:
                e = e + _EPS
            cov[(r, c)] = e

        l00 = jnp.sqrt(cov[(0, 0)]); i0 = 1.0 / l00
        l10 = cov[(1, 0)] * i0
        l20 = cov[(2, 0)] * i0
        l30 = cov[(3, 0)] * i0
        l11 = jnp.sqrt(cov[(1, 1)] - l10 * l10); i1 = 1.0 / l11
        l21 = (cov[(2, 1)] - l20 * l10) * i1
        l31 = (cov[(3, 1)] - l30 * l10) * i1
        l22 = jnp.sqrt(cov[(2, 2)] - l20 * l20 - l21 * l21); i2 = 1.0 / l22
        l32 = (cov[(3, 2)] - l30 * l20 - l31 * l21) * i2
        l33 = jnp.sqrt(cov[(3, 3)] - l30 * l30 - l31 * l31 - l32 * l32)
        i3 = 1.0 / l33

        mm = {}
        mm[(0, 0)] = i0; mm[(1, 1)] = i1; mm[(2, 2)] = i2; mm[(3, 3)] = i3
        mm[(1, 0)] = -(l10 * mm[(0, 0)]) * i1
        mm[(2, 0)] = -(l20 * mm[(0, 0)] + l21 * mm[(1, 0)]) * i2
        mm[(2, 1)] = -(l21 * mm[(1, 1)]) * i2
        mm[(3, 0)] = -(l30 * mm[(0, 0)] + l31 * mm[(1, 0)] + l32 * mm[(2, 0)]) * i3
        mm[(3, 1)] = -(l31 * mm[(1, 1)] + l32 * mm[(2, 1)]) * i3
        mm[(3, 2)] = -(l32 * mm[(2, 2)]) * i3

        g = {}
        for k, (r, c) in enumerate(zip(_TRIL_R, _TRIL_C)):
            g[(r, c)] = row(14 + k)
            g[(c, r)] = g[(r, c)]

        a = {}
        for i in range(4):
            for jj in range(4):
                acc = None
                for k in range(jj, 4):
                    t = g[(i, k)] * mm[(k, jj)]
                    acc = t if acc is None else acc + t
                a[(i, jj)] = acc

        bias = []
        for i in range(4):
            s = row(24 + i)
            for jj in range(4):
                s = s - a[(i, jj)] * m[jj]
            bias.append(s)

        lanes = p_ref.shape[1]
        out_rows = [a[(i, jj)] for i in range(4) for jj in range(4)]
        out_rows += bias
        out_rows += [jnp.zeros((1, lanes), jnp.float32)] * 4
        q_ref[...] = jnp.concatenate(out_rows, axis=0)

    return _solve_kernel


def _apply_kernel(cb_ref, x_ref, o_ref):
    xb = x_ref[...]
    lanes = xb.shape[1]
    acc = cb_ref[7:8, :] + cb_ref[3:4, :] * xb
    for k, delta in ((0, -3), (1, -2), (2, -1), (4, 1), (5, 2), (6, 3)):
        acc = acc + cb_ref[k:k + 1, :] * pltpu.roll(xb, (-delta) % lanes, axis=1)
    o_ref[...] = acc


def kernel(x, gamma, beta):
    batch, nfeat, dim = x.shape
    lanes = nfeat * dim
    n_inner = batch // (_CORES * _ROWS_BLK)
    xv = x.reshape(batch, lanes)
    f32 = jnp.float32

    cparams = pltpu.CompilerParams(
        dimension_semantics=("parallel", "arbitrary"))

    s_parts = pl.pallas_call(
        _stats_kernel,
        grid=(_CORES, n_inner),
        in_specs=[pl.BlockSpec((_ROWS_BLK, lanes),
                               lambda c, j: (c * n_inner + j, 0))],
        out_specs=pl.BlockSpec((1, 8, lanes), lambda c, j: (c, 0, 0)),
        out_shape=jax.ShapeDtypeStruct((_CORES, 8, lanes), f32),
        compiler_params=cparams,
    )(xv)
    ssum = s_parts[0] + s_parts[1]

    s1 = ssum[0].reshape(nfeat, dim).T
    s2 = ssum[1:5].reshape(4, nfeat, dim)
    covrows = jnp.stack(
        [s2[r - c, :, c] for r, c in zip(_TRIL_R, _TRIL_C)], axis=0)
    p_in = jnp.concatenate(
        [s1, covrows, gamma.T, beta.T, jnp.zeros((4, nfeat), f32)], axis=0)

    q = pl.pallas_call(
        _make_solve_kernel(batch),
        out_shape=jax.ShapeDtypeStruct((24, nfeat), f32),
    )(p_in)

    a_mat = q[:16].reshape(4, 4, nfeat)
    bias_r = q[16:20]
    cb_rows = []
    for delta in range(-3, 4):
        cols = [a_mat[i, i + delta] if 0 <= i + delta < 4
                else jnp.zeros((nfeat,), f32) for i in range(4)]
        cb_rows.append(jnp.stack(cols, axis=1).reshape(lanes))
    cb_rows.append(bias_r.T.reshape(lanes))
    cb = jnp.stack(cb_rows, axis=0)

    out = pl.pallas_call(
        _apply_kernel,
        grid=(_CORES, n_inner),
        in_specs=[
            pl.BlockSpec((8, lanes), lambda c, j: (0, 0)),
            pl.BlockSpec((_ROWS_BLK, lanes),
                         lambda c, j: (c * n_inner + j, 0)),
        ],
        out_specs=pl.BlockSpec((_ROWS_BLK, lanes),
                               lambda c, j: (c * n_inner + j, 0)),
        out_shape=jax.ShapeDtypeStruct((batch, lanes), f32),
        compiler_params=cparams,
    )(cb, xv)
    return out.reshape(batch, nfeat, dim)
